# Initial kernel scaffold; baseline (speedup 1.0000x reference)
import jax
import jax.numpy as jnp
from jax.experimental import pallas as pl


def kernel(text, emb_table, fc_w, fc_b):
    raise NotImplementedError("write your pallas kernel here")


# speedup vs baseline: 2.3753x; 2.3753x over previous
import functools

import jax
import jax.numpy as jnp
from jax import lax
from jax.experimental import pallas as pl
from jax.experimental.pallas import tpu as pltpu
from jax.experimental.pallas import tpu_sc as plsc

N_TOKENS = 1048576
EMBED_DIM = 32
NUM_CLASS = 4
BATCH = 16
SEG = N_TOKENS // BATCH
HALF = SEG // 2
CHUNK = 1024
SUBCHUNK = 128
N_SUB = CHUNK // SUBCHUNK
N_STEPS = HALF // CHUNK


def _seg_sum_kernel(text2d, emb_table):
    mesh = plsc.VectorSubcoreMesh(core_axis_name="c", subcore_axis_name="s")

    @functools.partial(
        pl.kernel,
        mesh=mesh,
        out_type=jax.ShapeDtypeStruct((32, EMBED_DIM), jnp.float32),
        scratch_types=[
            pltpu.VMEM((N_SUB, SUBCHUNK), jnp.int32),
            pltpu.VMEM((CHUNK, EMBED_DIM), jnp.float32),
            pltpu.VMEM((EMBED_DIM,), jnp.float32),
            pltpu.SemaphoreType.DMA,
        ],
        compiler_params=pltpu.CompilerParams(use_tc_tiling_on_sc=False),
    )
    def body(text_hbm, emb_hbm, out_hbm, idx_v, rows_v, acc_v, sem):
        c = lax.axis_index("c")
        s = lax.axis_index("s")
        wid = c * 16 + s
        row_base = s * (SEG // SUBCHUNK) + c * (HALF // SUBCHUNK)

        zero = jnp.zeros((16,), jnp.float32)

        def step(i, carry):
            a0, a1 = carry
            rb = row_base + i * N_SUB
            pltpu.sync_copy(text_hbm.at[pl.ds(rb, N_SUB)], idx_v)
            copies = []
            for j in range(N_SUB):
                copies.append(
                    pltpu.async_copy(
                        emb_hbm.at[idx_v.at[j]],
                        rows_v.at[pl.ds(j * SUBCHUNK, SUBCHUNK)],
                        sem,
                    )
                )
            for cp in copies:
                cp.wait()

            def rows8(r, carry2):
                b0, b1 = carry2
                base = r * 8
                for u in range(8):
                    b0 = b0 + rows_v[base + u, pl.ds(0, 16)]
                    b1 = b1 + rows_v[base + u, pl.ds(16, 16)]
                return b0, b1

            return lax.fori_loop(0, CHUNK // 8, rows8, (a0, a1))

        a0, a1 = lax.fori_loop(0, N_STEPS, step, (zero, zero))
        acc_v[pl.ds(0, 16)] = a0
        acc_v[pl.ds(16, 16)] = a1
        pltpu.sync_copy(acc_v, out_hbm.at[wid])

    return body(text2d, emb_table)


def _head_body(p_ref, w_ref, b_ref, o_ref):
    p = p_ref[...]
    pooled = (p[0:16, :] + p[16:32, :]) * (1.0 / SEG)
    o_ref[...] = (
        lax.dot_general(
            pooled, w_ref[...], (((1,), (1,)), ((), ())),
            preferred_element_type=jnp.float32,
        )
        + b_ref[...]
    )


def kernel(text, emb_table, fc_w, fc_b):
    text2d = text.reshape(N_TOKENS // SUBCHUNK, SUBCHUNK)
    partials = _seg_sum_kernel(text2d, emb_table)
    return pl.pallas_call(
        _head_body,
        out_shape=jax.ShapeDtypeStruct((BATCH, NUM_CLASS), jnp.float32),
    )(partials, fc_w, fc_b.reshape(1, NUM_CLASS))


# speedup vs baseline: 2.6421x; 1.1123x over previous
import functools

import jax
import jax.numpy as jnp
from jax import lax
from jax.experimental import pallas as pl
from jax.experimental.pallas import tpu as pltpu
from jax.experimental.pallas import tpu_sc as plsc

N_TOKENS = 1048576
EMBED_DIM = 32
NUM_CLASS = 4
BATCH = 16
SEG = N_TOKENS // BATCH
HALF = SEG // 2
CHUNK = 1024
SUBCHUNK = 128
N_SUB = CHUNK // SUBCHUNK
N_STEPS = HALF // CHUNK


def _seg_sum_kernel(text2d, emb_table):
    mesh = plsc.VectorSubcoreMesh(core_axis_name="c", subcore_axis_name="s")

    @functools.partial(
        pl.kernel,
        mesh=mesh,
        out_type=jax.ShapeDtypeStruct((32, EMBED_DIM), jnp.float32),
        scratch_types=[
            pltpu.VMEM((2, N_SUB, SUBCHUNK), jnp.int32),
            pltpu.VMEM((2, CHUNK, EMBED_DIM), jnp.float32),
            pltpu.VMEM((EMBED_DIM,), jnp.float32),
            pltpu.SemaphoreType.DMA,
            pltpu.SemaphoreType.DMA,
        ],
        compiler_params=pltpu.CompilerParams(use_tc_tiling_on_sc=False),
    )
    def body(text_hbm, emb_hbm, out_hbm, idx_v, rows_v, acc_v, isem, gsem):
        c = lax.axis_index("c")
        s = lax.axis_index("s")
        wid = c * 16 + s
        row_base = s * (SEG // SUBCHUNK) + c * (HALF // SUBCHUNK)
        last = N_STEPS - 1

        def idx_copy(step, b):
            rb = row_base + lax.min(step, last) * N_SUB
            return pltpu.make_async_copy(
                text_hbm.at[pl.ds(rb, N_SUB)], idx_v.at[b], isem
            )

        def gather(b, j):
            return pltpu.make_async_copy(
                emb_hbm.at[idx_v.at[b].at[j]],
                rows_v.at[b].at[pl.ds(j * SUBCHUNK, SUBCHUNK)],
                gsem,
            )

        idx_copy(0, 0).start()
        idx_copy(0, 0).wait()
        for j in range(N_SUB):
            gather(0, j).start()
        idx_copy(1, 1).start()

        zero = jnp.zeros((16,), jnp.float32)

        def accum(b, carry):
            def rows8(r, c2):
                b0, b1 = c2
                base = r * 8
                for u in range(8):
                    b0 = b0 + rows_v[b, base + u, pl.ds(0, 16)]
                    b1 = b1 + rows_v[b, base + u, pl.ds(16, 16)]
                return b0, b1

            return lax.fori_loop(0, CHUNK // 8, rows8, carry)

        def pair(g, carry):
            for b in (0, 1):
                i = g * 2 + b
                idx_copy(i + 1, 1 - b).wait()
                for j in range(N_SUB):
                    gather(1 - b, j).start()
                for j in range(N_SUB):
                    gather(b, j).wait()
                idx_copy(i + 2, b).start()
                carry = accum(b, carry)
            return carry

        a0, a1 = lax.fori_loop(0, N_STEPS // 2, pair, (zero, zero))

        idx_copy(N_STEPS + 1, 1).wait()
        for j in range(N_SUB):
            gather(0, j).wait()

        acc_v[pl.ds(0, 16)] = a0
        acc_v[pl.ds(16, 16)] = a1
        pltpu.sync_copy(acc_v, out_hbm.at[wid])

    return body(text2d, emb_table)


def _head_body(p_ref, w_ref, b_ref, o_ref):
    p = p_ref[...]
    pooled = (p[0:16, :] + p[16:32, :]) * (1.0 / SEG)
    o_ref[...] = (
        lax.dot_general(
            pooled, w_ref[...], (((1,), (1,)), ((), ())),
            preferred_element_type=jnp.float32,
        )
        + b_ref[...]
    )


def kernel(text, emb_table, fc_w, fc_b):
    text2d = text.reshape(N_TOKENS // SUBCHUNK, SUBCHUNK)
    partials = _seg_sum_kernel(text2d, emb_table)
    return pl.pallas_call(
        _head_body,
        out_shape=jax.ShapeDtypeStruct((BATCH, NUM_CLASS), jnp.float32),
    )(partials, fc_w, fc_b.reshape(1, NUM_CLASS))


# speedup vs baseline: 4.4528x; 1.6853x over previous
import functools

import jax
import jax.numpy as jnp
from jax import lax
from jax.experimental import pallas as pl
from jax.experimental.pallas import tpu as pltpu
from jax.experimental.pallas import tpu_sc as plsc

N_TOKENS = 1048576
VOCAB = 1000000
EMBED_DIM = 32
VOCABX = VOCAB * EMBED_DIM
NUM_CLASS = 4
BATCH = 16
SEG = N_TOKENS // BATCH
HALF = SEG // 2
CHUNK = 1024
SUBCHUNK = 128
N_SUB = CHUNK // SUBCHUNK
N_STEPS = HALF // CHUNK


def _seg_sum_kernel(text2d, emb_table):
    mesh = plsc.VectorSubcoreMesh(core_axis_name="c", subcore_axis_name="s")

    @functools.partial(
        pl.kernel,
        mesh=mesh,
        out_type=jax.ShapeDtypeStruct((32, EMBED_DIM), jnp.float32),
        scratch_types=[
            pltpu.VMEM((2, N_SUB, SUBCHUNK), jnp.int32),
            pltpu.VMEM((2, CHUNK, EMBED_DIM), jnp.float32),
            pltpu.VMEM((EMBED_DIM,), jnp.float32),
            pltpu.SemaphoreType.DMA,
            pltpu.SemaphoreType.DMA,
        ],
        compiler_params=pltpu.CompilerParams(use_tc_tiling_on_sc=False),
    )
    def body(text_hbm, emb_hbm, out_hbm, idx_v, rows_v, acc_v, isem, gsem):
        c = lax.axis_index("c")
        s = lax.axis_index("s")
        wid = c * 16 + s
        row_base = s * (SEG // SUBCHUNK) + c * (HALF // SUBCHUNK)
        last = N_STEPS - 1

        def idx_copy(step, b):
            rb = row_base + lax.min(step, last) * N_SUB
            return pltpu.make_async_copy(
                text_hbm.at[pl.ds(rb, N_SUB)], idx_v.at[b], isem
            )

        def gather(b, j):
            return pltpu.make_async_copy(
                emb_hbm.at[idx_v.at[b].at[j]],
                rows_v.at[b].at[pl.ds(j * SUBCHUNK, SUBCHUNK)],
                gsem,
            )

        idx_copy(0, 0).start()
        idx_copy(0, 0).wait()
        for j in range(N_SUB):
            gather(0, j).start()
        idx_copy(1, 1).start()

        zero = jnp.zeros((16,), jnp.float32)

        def accum(b, carry):
            def rows8(r, c2):
                b0, b1 = c2
                base = r * 8
                for u in range(8):
                    b0 = b0 + rows_v[b, base + u, pl.ds(0, 16)]
                    b1 = b1 + rows_v[b, base + u, pl.ds(16, 16)]
                return b0, b1

            return lax.fori_loop(0, CHUNK // 8, rows8, carry)

        def pair(g, carry):
            for b in (0, 1):
                i = g * 2 + b
                idx_copy(i + 1, 1 - b).wait()
                for j in range(N_SUB):
                    gather(1 - b, j).start()
                for j in range(N_SUB):
                    gather(b, j).wait()
                idx_copy(i + 2, b).start()
                carry = accum(b, carry)
            return carry

        a0, a1 = lax.fori_loop(0, N_STEPS // 2, pair, (zero, zero))

        idx_copy(N_STEPS + 1, 1).wait()
        for j in range(N_SUB):
            gather(0, j).wait()

        acc_v[pl.ds(0, 16)] = a0
        acc_v[pl.ds(16, 16)] = a1
        pltpu.sync_copy(acc_v, out_hbm.at[wid])

    return body(text2d, emb_table)


TR_BLK = 2048
TR_I = 123
SLAB = TR_BLK * TR_I


def _transpose_body(t0, t1, t2, t3, o_ref):
    o_ref[...] = jnp.concatenate(
        [t0[...].T, t1[...].T, t2[...].T, t3[...].T], axis=1
    )


def _linearize_table(emb_table):
    emb_t = emb_table.T
    lin = pl.pallas_call(
        _transpose_body,
        grid=(TR_I,),
        in_specs=[
            pl.BlockSpec(
                (EMBED_DIM, TR_BLK),
                lambda i, j=j: (0, jnp.minimum(j * TR_I + i, VOCAB // TR_BLK)),
            )
            for j in range(4)
        ],
        out_specs=pl.BlockSpec((TR_BLK, 4 * EMBED_DIM), lambda i: (i, 0)),
        out_shape=jax.ShapeDtypeStruct((SLAB, 4 * EMBED_DIM), jnp.float32),
    )(emb_t, emb_t, emb_t, emb_t)
    return lin.reshape(4 * SLAB, EMBED_DIM)


def _head_body(p_ref, w_ref, b_ref, o_ref):
    p = p_ref[...]
    pooled = (p[0:16, :] + p[16:32, :]) * (1.0 / SEG)
    o_ref[...] = (
        lax.dot_general(
            pooled, w_ref[...], (((1,), (1,)), ((), ())),
            preferred_element_type=jnp.float32,
        )
        + b_ref[...]
    )


def kernel(text, emb_table, fc_w, fc_b):
    emb_lin = _linearize_table(emb_table)
    slab_id = (
        (text >= SLAB).astype(jnp.int32)
        + (text >= 2 * SLAB).astype(jnp.int32)
        + (text >= 3 * SLAB).astype(jnp.int32)
    )
    text_r = 4 * text - slab_id * (4 * SLAB - 1)
    text2d = text_r.reshape(N_TOKENS // SUBCHUNK, SUBCHUNK)
    partials = _seg_sum_kernel(text2d, emb_lin)
    return pl.pallas_call(
        _head_body,
        out_shape=jax.ShapeDtypeStruct((BATCH, NUM_CLASS), jnp.float32),
    )(partials, fc_w, fc_b.reshape(1, NUM_CLASS))


# speedup vs baseline: 6.8047x; 1.5282x over previous
import functools

import jax
import jax.numpy as jnp
from jax import lax
from jax.experimental import pallas as pl
from jax.experimental.pallas import tpu as pltpu
from jax.experimental.pallas import tpu_sc as plsc

N_TOKENS = 1048576
VOCAB = 1000000
EMBED_DIM = 32
VOCABX = VOCAB * EMBED_DIM
NUM_CLASS = 4
BATCH = 16
SEG = N_TOKENS // BATCH
HALF = SEG // 2
CHUNK = 1024
SUBCHUNK = 128
N_SUB = CHUNK // SUBCHUNK
N_STEPS = HALF // CHUNK


def _seg_sum_kernel(text2d, emb_table):
    mesh = plsc.VectorSubcoreMesh(core_axis_name="c", subcore_axis_name="s")

    @functools.partial(
        pl.kernel,
        mesh=mesh,
        out_type=jax.ShapeDtypeStruct((32, EMBED_DIM), jnp.float32),
        scratch_types=[
            pltpu.VMEM((2, N_SUB, SUBCHUNK), jnp.int32),
            pltpu.VMEM((2, CHUNK, EMBED_DIM), jnp.float32),
            pltpu.VMEM((EMBED_DIM,), jnp.float32),
            pltpu.SemaphoreType.DMA,
            pltpu.SemaphoreType.DMA,
        ],
        compiler_params=pltpu.CompilerParams(use_tc_tiling_on_sc=False),
    )
    def body(text_hbm, emb_hbm, out_hbm, idx_v, rows_v, acc_v, isem, gsem):
        c = lax.axis_index("c")
        s = lax.axis_index("s")
        wid = c * 16 + s
        row_base = s * (SEG // SUBCHUNK) + c * (HALF // SUBCHUNK)
        last = N_STEPS - 1

        def idx_copy(step, b):
            rb = row_base + lax.min(step, last) * N_SUB
            return pltpu.make_async_copy(
                text_hbm.at[pl.ds(rb, N_SUB)], idx_v.at[b], isem
            )

        def gather(b, j):
            return pltpu.make_async_copy(
                emb_hbm.at[idx_v.at[b].at[j]],
                rows_v.at[b].at[pl.ds(j * SUBCHUNK, SUBCHUNK)],
                gsem,
            )

        idx_copy(0, 0).start()
        idx_copy(0, 0).wait()
        for j in range(N_SUB):
            gather(0, j).start()
        idx_copy(1, 1).start()

        zero = jnp.zeros((16,), jnp.float32)

        def accum(b, carry):
            def rows8(r, c2):
                b0, b1 = c2
                base = r * 8
                for u in range(8):
                    b0 = b0 + rows_v[b, base + u, pl.ds(0, 16)]
                    b1 = b1 + rows_v[b, base + u, pl.ds(16, 16)]
                return b0, b1

            return lax.fori_loop(0, CHUNK // 8, rows8, carry)

        def pair(g, carry):
            for b in (0, 1):
                i = g * 2 + b
                idx_copy(i + 1, 1 - b).wait()
                for j in range(N_SUB):
                    gather(1 - b, j).start()
                for j in range(N_SUB):
                    gather(b, j).wait()
                idx_copy(i + 2, b).start()
                carry = accum(b, carry)
            return carry

        a0, a1 = lax.fori_loop(0, N_STEPS // 2, pair, (zero, zero))

        idx_copy(N_STEPS + 1, 1).wait()
        for j in range(N_SUB):
            gather(0, j).wait()

        acc_v[pl.ds(0, 16)] = a0
        acc_v[pl.ds(16, 16)] = a1
        pltpu.sync_copy(acc_v, out_hbm.at[wid])

    return body(text2d, emb_table)


TR_BLK = 2048
TR_I = 123
SLAB = TR_BLK * TR_I


def _transpose_body(t0, t1, t2, t3, o_ref):
    tcat = jnp.concatenate([t0[...], t1[...], t2[...], t3[...]], axis=0)
    n = 4 * EMBED_DIM
    rows = lax.broadcasted_iota(jnp.int32, (n, n), 0)
    cols = lax.broadcasted_iota(jnp.int32, (n, n), 1)
    ident = (cols == rows).astype(jnp.float32)
    o_ref[...] = lax.dot_general(
        tcat, ident, (((0,), (0,)), ((), ())),
        preferred_element_type=jnp.float32,
    )


def _linearize_table(emb_table):
    emb_t = emb_table.T
    lin = pl.pallas_call(
        _transpose_body,
        grid=(TR_I,),
        in_specs=[
            pl.BlockSpec(
                (EMBED_DIM, TR_BLK),
                lambda i, j=j: (0, jnp.minimum(j * TR_I + i, VOCAB // TR_BLK)),
            )
            for j in range(4)
        ],
        out_specs=pl.BlockSpec((TR_BLK, 4 * EMBED_DIM), lambda i: (i, 0)),
        out_shape=jax.ShapeDtypeStruct((SLAB, 4 * EMBED_DIM), jnp.float32),
    )(emb_t, emb_t, emb_t, emb_t)
    return lin.reshape(4 * SLAB, EMBED_DIM)


def _head_body(p_ref, w_ref, b_ref, o_ref):
    p = p_ref[...]
    pooled = (p[0:16, :] + p[16:32, :]) * (1.0 / SEG)
    o_ref[...] = (
        lax.dot_general(
            pooled, w_ref[...], (((1,), (1,)), ((), ())),
            preferred_element_type=jnp.float32,
        )
        + b_ref[...]
    )


def kernel(text, emb_table, fc_w, fc_b):
    emb_lin = _linearize_table(emb_table)
    slab_id = (
        (text >= SLAB).astype(jnp.int32)
        + (text >= 2 * SLAB).astype(jnp.int32)
        + (text >= 3 * SLAB).astype(jnp.int32)
    )
    text_r = 4 * text - slab_id * (4 * SLAB - 1)
    text2d = text_r.reshape(N_TOKENS // SUBCHUNK, SUBCHUNK)
    partials = _seg_sum_kernel(text2d, emb_lin)
    return pl.pallas_call(
        _head_body,
        out_shape=jax.ShapeDtypeStruct((BATCH, NUM_CLASS), jnp.float32),
    )(partials, fc_w, fc_b.reshape(1, NUM_CLASS))
